# speedup vs baseline: 1.4274x; 1.4274x over previous
import functools

import jax
import jax.numpy as jnp
from jax import lax
from jax.experimental import pallas as pl
from jax.experimental.pallas import tpu as pltpu

N = 10000
E = 320000
IN_CH = 128
PROJ = 64
C = 32
HEADS = 2
NUM_EDGE_TYPES = 8
ETE_DIM = 16
EAE_DIM = 16

BE = 5000
GE = E // BE


def _lr(x, s=0.2):
    return jnp.where(x >= 0, x, s * x)


def _p1_body(ea_ref, et_ref, W_ref, c_ref, M_ref, T_ref,
             eatt_ref, em_ref, mx_ref):
    step = pl.program_id(0)
    A = ea_ref[...]
    u = _lr(jnp.dot(A, W_ref[...], preferred_element_type=jnp.float32)
            + c_ref[...])
    types = et_ref[0, 0]
    oh = (types[:, None] ==
          lax.broadcasted_iota(jnp.int32, (BE, NUM_EDGE_TYPES), 1)
          ).astype(jnp.float32)
    t_att = jnp.dot(oh, T_ref[...], preferred_element_type=jnp.float32)
    cols = []
    for l in range(3):
        ul = u[:, 16 * l:16 * l + 16]
        o = jnp.dot(ul, M_ref[l], preferred_element_type=jnp.float32)
        em_ref[l, :, :] = o[:, 2:34]
        cols.append(o[:, 0:2] + t_att[:, 2 * l:2 * l + 2])
    eatt = jnp.concatenate(cols + [jnp.zeros((BE, 2), jnp.float32)], axis=1)
    eatt_ref[...] = eatt
    bmax = jnp.max(eatt, axis=0, keepdims=True)

    @pl.when(step == 0)
    def _():
        mx_ref[...] = bmax

    @pl.when(step != 0)
    def _():
        mx_ref[...] = jnp.maximum(mx_ref[...], bmax)


def _p1(edge_attr, edge_types, W_all, c_all, M_all, T_all):
    et3 = edge_types.reshape(GE, 1, BE)
    return pl.pallas_call(
        _p1_body,
        grid=(GE,),
        in_specs=[
            pl.BlockSpec((BE, IN_CH), lambda i: (i, 0)),
            pl.BlockSpec((1, 1, BE), lambda i: (i, 0, 0)),
            pl.BlockSpec((IN_CH, 48), lambda i: (0, 0)),
            pl.BlockSpec((1, 48), lambda i: (0, 0)),
            pl.BlockSpec((3, 16, 34), lambda i: (0, 0, 0)),
            pl.BlockSpec((NUM_EDGE_TYPES, 6), lambda i: (0, 0)),
        ],
        out_specs=[
            pl.BlockSpec((BE, 8), lambda i: (i, 0)),
            pl.BlockSpec((3, BE, 32), lambda i: (0, i, 0)),
            pl.BlockSpec((1, 8), lambda i: (0, 0)),
        ],
        out_shape=[
            jax.ShapeDtypeStruct((E, 8), jnp.float32),
            jax.ShapeDtypeStruct((3, E, 32), jnp.float32),
            jax.ShapeDtypeStruct((1, 8), jnp.float32),
        ],
    )(edge_attr, et3, W_all, c_all, M_all, T_all)


def kernel(x, edge_index, edge_types, edge_attr, params):
    src, dst = edge_index[0], edge_index[1]
    proj_w, proj_b = params["proj_w"], params["proj_b"]

    folded = []
    for l in range(3):
        p = params["convs"][l]
        att_w = p["att_w"]
        folded.append(dict(
            hl_w0=p["hl_w"][0], hl_b0=p["hl_b"][0],
            att_wi=att_w[:32], att_wj=att_w[32:64],
            t_att=_lr(p["ete"]) @ att_w[64:80],
            att_wa=att_w[80:96],
            lin_n=p["lin_w"][:32], lin_e=p["lin_w"][32:48],
            W=proj_w @ p["eae_w"],
            c=proj_b @ p["eae_w"],
        ))

    W_all = jnp.concatenate([f["W"] for f in folded], axis=1)
    c_all = jnp.concatenate([f["c"] for f in folded])[None, :]
    M_all = jnp.stack([jnp.concatenate([f["att_wa"], f["lin_e"]], axis=1)
                       for f in folded])
    T_all = jnp.concatenate([f["t_att"] for f in folded], axis=1)

    e_att, e_m, mx = _p1(edge_attr, edge_types, W_all, c_all, M_all, T_all)
    maxE = mx[0]

    xw0_w = proj_w @ folded[0]["hl_w0"]
    xw0_b = proj_b @ folded[0]["hl_w0"] + folded[0]["hl_b0"]

    h = None
    for l in range(3):
        f = folded[l]
        if l == 0:
            xw = x @ xw0_w + xw0_b
        else:
            xw = h @ f["hl_w0"] + f["hl_b0"]
        ai = xw @ f["att_wi"]
        aj = xw @ f["att_wj"]
        n_m = xw @ f["lin_n"]
        bound = _lr(jnp.max(ai, 0) + jnp.max(aj, 0) + maxE[2 * l:2 * l + 2])

        alpha = _lr(ai[dst] + aj[src] + e_att[:, 2 * l:2 * l + 2])
        ex = jnp.exp(alpha - bound)
        m = n_m[src] + e_m[l]
        contrib = (ex[:, :, None] * m[:, None, :]).reshape(E, 64)
        S = jax.ops.segment_sum(contrib, dst, num_segments=N)
        D = jax.ops.segment_sum(ex, dst, num_segments=N)

        agg = S.reshape(N, 2, 32) / (D[:, :, None] + 1e-16) + xw[:, None, :]
        hn = agg.reshape(N, 64)
        if l < 2:
            bn = params["bns"][l]
            mu = hn.mean(0)
            var = hn.var(0)
            hn = bn["g"] * (hn - mu) / jnp.sqrt(var + 1e-5) + bn["b"]
            hn = jax.nn.relu(hn)
        h = hn
    return h.reshape(N, 2, 32)


# speedup vs baseline: 3.6477x; 2.5555x over previous
import functools

import jax
import jax.numpy as jnp
from jax import lax
from jax.experimental import pallas as pl
from jax.experimental.pallas import tpu as pltpu
from jax.experimental.pallas import tpu_sc as plsc

N = 10000
E = 320000
E2 = E // 2
IN_CH = 128
PROJ = 64
C = 32
HEADS = 2
NUM_EDGE_TYPES = 8

BE2 = 2000
GE = E2 // BE2


def _lr(x, s=0.2):
    return jnp.where(x >= 0, x, s * x)


def _p1_half(A, types, W_ref, c_ref, M_ref, T_ref):
    u = _lr(jnp.dot(A, W_ref[...], preferred_element_type=jnp.float32)
            + c_ref[...])
    oh = (types[:, None] ==
          lax.broadcasted_iota(jnp.int32, (BE2, NUM_EDGE_TYPES), 1)
          ).astype(jnp.float32)
    t_att = jnp.dot(oh, T_ref[...], preferred_element_type=jnp.float32)
    ems, eatts = [], []
    for l in range(3):
        ul = u[:, 16 * l:16 * l + 16]
        o = jnp.dot(ul, M_ref[l], preferred_element_type=jnp.float32)
        ems.append(o[:, 2:34])
        eatts.append(o[:, 0:2] + t_att[:, 2 * l:2 * l + 2])
    return ems, eatts


def _p1_body(eaA_ref, eaB_ref, etA_ref, etB_ref, W_ref, c_ref, M_ref, T_ref,
             edata_ref, mx_ref):
    step = pl.program_id(0)
    emsA, eattsA = _p1_half(eaA_ref[...], etA_ref[0, 0], W_ref, c_ref, M_ref, T_ref)
    emsB, eattsB = _p1_half(eaB_ref[...], etB_ref[0, 0], W_ref, c_ref, M_ref, T_ref)
    bm = []
    for l in range(3):
        edata_ref[l, :, 0:32] = emsA[l]
        edata_ref[l, :, 32:34] = eattsA[l]
        edata_ref[l, :, 64:96] = emsB[l]
        edata_ref[l, :, 96:98] = eattsB[l]
        bm.append(jnp.maximum(jnp.max(eattsA[l], axis=0), jnp.max(eattsB[l], axis=0)))
    bmax = jnp.concatenate(bm + [jnp.zeros((2,), jnp.float32)])[None, :]

    @pl.when(step == 0)
    def _():
        mx_ref[...] = bmax

    @pl.when(step != 0)
    def _():
        mx_ref[...] = jnp.maximum(mx_ref[...], bmax)


def _p1(edge_attr, edge_types, W_all, c_all, M_all, T_all):
    etA = edge_types[:E2].reshape(GE, 1, BE2)
    etB = edge_types[E2:].reshape(GE, 1, BE2)
    return pl.pallas_call(
        _p1_body,
        grid=(GE,),
        in_specs=[
            pl.BlockSpec((BE2, IN_CH), lambda i: (i, 0)),
            pl.BlockSpec((BE2, IN_CH), lambda i: (i + GE, 0)),
            pl.BlockSpec((1, 1, BE2), lambda i: (i, 0, 0)),
            pl.BlockSpec((1, 1, BE2), lambda i: (i, 0, 0)),
            pl.BlockSpec((IN_CH, 48), lambda i: (0, 0)),
            pl.BlockSpec((1, 48), lambda i: (0, 0)),
            pl.BlockSpec((3, 16, 34), lambda i: (0, 0, 0)),
            pl.BlockSpec((NUM_EDGE_TYPES, 6), lambda i: (0, 0)),
        ],
        out_specs=[
            pl.BlockSpec((3, BE2, 128), lambda i: (0, i, 0)),
            pl.BlockSpec((1, 8), lambda i: (0, 0)),
        ],
        out_shape=[
            jax.ShapeDtypeStruct((3, E2, 128), jnp.float32),
            jax.ShapeDtypeStruct((1, 8), jnp.float32),
        ],
    )(edge_attr, edge_attr, etA, etB, W_all, c_all, M_all, T_all)


NC = 2
NS = 16
NW = NC * NS
RPW = E2 // NW
RCH = 40
NCHUNK = RPW // RCH
CH = 2 * RCH
ROWW = 128
NPS = 632
NPAD = NS * NPS


def _sc_edge_body(L):
    def body(srcA, srcB, dstA, dstB, edata, nmp, tabf, cvec, zrows, out,
             Sacc, tab_v, cvec_v, src_v, dst_v, ed_v, nm_v, out_v,
             sem_g, sem_s):
        core = lax.axis_index("c")
        sub = lax.axis_index("s")
        wid = core * NS + sub
        rbase0 = wid * RPW

        pltpu.sync_copy(zrows, Sacc.at[pl.ds(sub * NPS, NPS)])
        pltpu.sync_copy(tabf, tab_v)
        pltpu.sync_copy(cvec, cvec_v)
        plsc.subcore_barrier()

        def chunk(k, _):
            lanes0 = lax.iota(jnp.int32, 16)
            c0 = cvec_v[pl.ds(0, 16)]
            c1 = cvec_v[pl.ds(16, 16)]
            rbase = rbase0 + k * RCH
            pltpu.sync_copy(srcA.at[pl.ds(rbase, RCH)], src_v.at[pl.ds(0, RCH)])
            pltpu.sync_copy(srcB.at[pl.ds(rbase, RCH)], src_v.at[pl.ds(RCH, RCH)])
            pltpu.sync_copy(dstA.at[pl.ds(rbase, RCH)], dst_v.at[pl.ds(0, RCH)])
            pltpu.sync_copy(dstB.at[pl.ds(rbase, RCH)], dst_v.at[pl.ds(RCH, RCH)])
            pltpu.sync_copy(edata.at[L, pl.ds(rbase, RCH)], ed_v)
            gA = pltpu.async_copy(nmp.at[src_v.at[pl.ds(0, RCH)]],
                                  nm_v.at[pl.ds(0, RCH)], sem_g)
            gB = pltpu.async_copy(nmp.at[src_v.at[pl.ds(RCH, RCH)]],
                                  nm_v.at[pl.ds(RCH, RCH)], sem_g)
            gA.wait()
            gB.wait()
            for g in range(CH // 16):
                lanes = lanes0 + g * 16
                dstg = dst_v[pl.ds(g * 16, 16)]
                isB = (lanes >= RCH).astype(jnp.int32)
                edrow = lanes - isB * RCH
                colb = isB * 64
                ai0 = plsc.load_gather(tab_v, [dstg])
                ai1 = plsc.load_gather(tab_v, [dstg + N])
                aj0 = plsc.load_gather(nm_v, [lanes, jnp.full((16,), C, jnp.int32)])
                aj1 = plsc.load_gather(nm_v, [lanes, jnp.full((16,), C + 1, jnp.int32)])
                e0 = plsc.load_gather(ed_v, [edrow, colb + 32])
                e1 = plsc.load_gather(ed_v, [edrow, colb + 33])
                ex0 = jnp.exp(_lr(ai0 + aj0 + e0) - c0)
                ex1 = jnp.exp(_lr(ai1 + aj1 + e1) - c1)
                for c2 in range(C):
                    fc = jnp.full((16,), c2, jnp.int32)
                    m = (plsc.load_gather(ed_v, [edrow, colb + c2])
                         + plsc.load_gather(nm_v, [lanes, fc]))
                    plsc.store_scatter(out_v, [lanes, fc], m * ex0)
                    plsc.store_scatter(out_v, [lanes, jnp.full((16,), C + c2, jnp.int32)],
                                       m * ex1)
                plsc.store_scatter(out_v, [lanes, jnp.full((16,), 2 * C, jnp.int32)], ex0)
                plsc.store_scatter(out_v, [lanes, jnp.full((16,), 2 * C + 1, jnp.int32)], ex1)
            pltpu.async_copy(out_v, Sacc.at[dst_v], sem_s, add=True).wait()
            return ()

        lax.fori_loop(0, NCHUNK, chunk, ())
        plsc.subcore_barrier()
        pltpu.sync_copy(Sacc.at[pl.ds(sub * NPS, NPS)], out.at[wid])

    return body


def _sc_edge(L, srcA, srcB, dstA, dstB, edata, nmp, tabf, cvec, zrows):
    f = pl.kernel(
        _sc_edge_body(L),
        out_type=jax.ShapeDtypeStruct((NW, NPS, ROWW), jnp.float32),
        mesh=plsc.VectorSubcoreMesh(core_axis_name="c", subcore_axis_name="s"),
        compiler_params=pltpu.CompilerParams(needs_layout_passes=False),
        scratch_types=[
            pltpu.VMEM_SHARED((NPAD, ROWW), jnp.float32),
            pltpu.VMEM((2 * N,), jnp.float32),
            pltpu.VMEM((32,), jnp.float32),
            pltpu.VMEM((CH,), jnp.int32),
            pltpu.VMEM((CH,), jnp.int32),
            pltpu.VMEM((RCH, 128), jnp.float32),
            pltpu.VMEM((CH, 128), jnp.float32),
            pltpu.VMEM((CH, ROWW), jnp.float32),
            pltpu.SemaphoreType.DMA,
            pltpu.SemaphoreType.DMA,
        ],
    )
    return f(srcA, srcB, dstA, dstB, edata, nmp, tabf, cvec, zrows)


def kernel(x, edge_index, edge_types, edge_attr, params):
    src, dst = edge_index[0], edge_index[1]
    srcA, srcB = src[:E2], src[E2:]
    dstA, dstB = dst[:E2], dst[E2:]
    proj_w, proj_b = params["proj_w"], params["proj_b"]

    folded = []
    for l in range(3):
        p = params["convs"][l]
        att_w = p["att_w"]
        folded.append(dict(
            hl_w0=p["hl_w"][0], hl_b0=p["hl_b"][0],
            att_wi=att_w[:32], att_wj=att_w[32:64],
            t_att=_lr(p["ete"]) @ att_w[64:80],
            att_wa=att_w[80:96],
            lin_n=p["lin_w"][:32], lin_e=p["lin_w"][32:48],
            W=proj_w @ p["eae_w"],
            c=proj_b @ p["eae_w"],
        ))

    W_all = jnp.concatenate([f["W"] for f in folded], axis=1)
    c_all = jnp.concatenate([f["c"] for f in folded])[None, :]
    M_all = jnp.stack([jnp.concatenate([f["att_wa"], f["lin_e"]], axis=1)
                       for f in folded])
    T_all = jnp.concatenate([f["t_att"] for f in folded], axis=1)

    edata, mx = _p1(edge_attr, edge_types, W_all, c_all, M_all, T_all)
    maxE = mx[0]

    xw0_w = proj_w @ folded[0]["hl_w0"]
    xw0_b = proj_b @ folded[0]["hl_w0"] + folded[0]["hl_b0"]
    zrows = jnp.zeros((NPS, ROWW), jnp.float32)

    h = None
    for l in range(3):
        f = folded[l]
        if l == 0:
            xw = x @ xw0_w + xw0_b
        else:
            xw = h @ f["hl_w0"] + f["hl_b0"]
        ai = xw @ f["att_wi"]
        aj = xw @ f["att_wj"]
        n_m = xw @ f["lin_n"]
        bound = _lr(jnp.max(ai, 0) + jnp.max(aj, 0) + maxE[2 * l:2 * l + 2])

        tabf = jnp.concatenate([ai[:, 0], ai[:, 1]])
        cvec = jnp.repeat(bound, 16)
        nmp = jnp.concatenate(
            [n_m, aj, jnp.zeros((N, 128 - C - 2), jnp.float32)], axis=1)

        Spart = _sc_edge(l, srcA, srcB, dstA, dstB, edata, nmp, tabf,
                         cvec, zrows)
        Spart = Spart.reshape(NC, NPAD, ROWW)
        Ssum = (Spart[0] + Spart[1])[:N]
        D = Ssum[:, 2 * C:2 * C + 2]
        agg = (Ssum[:, :2 * C].reshape(N, 2, 32)
               / (D[:, :, None] + 1e-16) + xw[:, None, :])
        hn = agg.reshape(N, 64)
        if l < 2:
            bn = params["bns"][l]
            mu = hn.mean(0)
            var = hn.var(0)
            hn = bn["g"] * (hn - mu) / jnp.sqrt(var + 1e-5) + bn["b"]
            hn = jax.nn.relu(hn)
        h = hn
    return h.reshape(N, 2, 32)


# speedup vs baseline: 4.5003x; 1.2337x over previous
import functools

import jax
import jax.numpy as jnp
from jax import lax
from jax.experimental import pallas as pl
from jax.experimental.pallas import tpu as pltpu
from jax.experimental.pallas import tpu_sc as plsc

N = 10000
E = 320000
E2 = E // 2
IN_CH = 128
PROJ = 64
C = 32
HEADS = 2
NUM_EDGE_TYPES = 8

BE2 = 2000
GE = E2 // BE2


def _lr(x, s=0.2):
    return jnp.where(x >= 0, x, s * x)


def _p1_half(A, types, W_ref, c_ref, M_ref, T_ref):
    u = _lr(jnp.dot(A, W_ref[...], preferred_element_type=jnp.float32)
            + c_ref[...])
    oh = (types[:, None] ==
          lax.broadcasted_iota(jnp.int32, (BE2, NUM_EDGE_TYPES), 1)
          ).astype(jnp.float32)
    t_att = jnp.dot(oh, T_ref[...], preferred_element_type=jnp.float32)
    ems, eatts = [], []
    for l in range(3):
        ul = u[:, 16 * l:16 * l + 16]
        o = jnp.dot(ul, M_ref[l], preferred_element_type=jnp.float32)
        ems.append(o[:, 2:34])
        eatts.append(o[:, 0:2] + t_att[:, 2 * l:2 * l + 2])
    return ems, eatts


def _p1_body(eaA_ref, eaB_ref, etA_ref, etB_ref, W_ref, c_ref, M_ref, T_ref,
             edata_ref, mx_ref):
    step = pl.program_id(0)
    emsA, eattsA = _p1_half(eaA_ref[...], etA_ref[0, 0], W_ref, c_ref, M_ref, T_ref)
    emsB, eattsB = _p1_half(eaB_ref[...], etB_ref[0, 0], W_ref, c_ref, M_ref, T_ref)
    bm = []
    for l in range(3):
        edata_ref[l, :, 0:32] = emsA[l]
        edata_ref[l, :, 32:34] = eattsA[l]
        edata_ref[l, :, 64:96] = emsB[l]
        edata_ref[l, :, 96:98] = eattsB[l]
        bm.append(jnp.maximum(jnp.max(eattsA[l], axis=0), jnp.max(eattsB[l], axis=0)))
    bmax = jnp.concatenate(bm + [jnp.zeros((2,), jnp.float32)])[None, :]

    @pl.when(step == 0)
    def _():
        mx_ref[...] = bmax

    @pl.when(step != 0)
    def _():
        mx_ref[...] = jnp.maximum(mx_ref[...], bmax)


def _p1(edge_attr, edge_types, W_all, c_all, M_all, T_all):
    etA = edge_types[:E2].reshape(GE, 1, BE2)
    etB = edge_types[E2:].reshape(GE, 1, BE2)
    return pl.pallas_call(
        _p1_body,
        grid=(GE,),
        in_specs=[
            pl.BlockSpec((BE2, IN_CH), lambda i: (i, 0)),
            pl.BlockSpec((BE2, IN_CH), lambda i: (i + GE, 0)),
            pl.BlockSpec((1, 1, BE2), lambda i: (i, 0, 0)),
            pl.BlockSpec((1, 1, BE2), lambda i: (i, 0, 0)),
            pl.BlockSpec((IN_CH, 48), lambda i: (0, 0)),
            pl.BlockSpec((1, 48), lambda i: (0, 0)),
            pl.BlockSpec((3, 16, 34), lambda i: (0, 0, 0)),
            pl.BlockSpec((NUM_EDGE_TYPES, 6), lambda i: (0, 0)),
        ],
        out_specs=[
            pl.BlockSpec((3, BE2, 128), lambda i: (0, i, 0)),
            pl.BlockSpec((1, 8), lambda i: (0, 0)),
        ],
        out_shape=[
            jax.ShapeDtypeStruct((3, E2, 128), jnp.float32),
            jax.ShapeDtypeStruct((1, 8), jnp.float32),
        ],
    )(edge_attr, edge_attr, etA, etB, W_all, c_all, M_all, T_all)


NC = 2
NS = 16
NW = NC * NS
RPW = E2 // NW
RCH = 40
NCHUNK = RPW // RCH
CH = 2 * RCH
ROWW = 128
NPS = 632
NPAD = NS * NPS


def _sc_edge_body(L):
    def body(srcA, srcB, dstA, dstB, edata, nmp, tabf, cvec, zrows, out,
             Sacc, tab_v, cvec_v, src_v, dst_v, ed_v, nm_v, out_v,
             sem_in, sem_g, sem_s):
        core = lax.axis_index("c")
        sub = lax.axis_index("s")
        wid = core * NS + sub
        rbase0 = wid * RPW

        pltpu.sync_copy(zrows, Sacc.at[pl.ds(sub * NPS, NPS)])
        pltpu.sync_copy(tabf, tab_v)
        pltpu.sync_copy(cvec, cvec_v)
        plsc.subcore_barrier()

        def chunk(k, _):
            lanes0 = lax.iota(jnp.int32, 16)
            c0 = cvec_v[pl.ds(0, 16)]
            c1 = cvec_v[pl.ds(16, 16)]
            rbase = rbase0 + k * RCH
            b = lax.rem(k, 2)
            bv = jnp.zeros((16,), jnp.int32) + b
            d1 = pltpu.make_async_copy(srcA.at[pl.ds(rbase, RCH)],
                                       src_v.at[pl.ds(0, RCH)], sem_in)
            d2 = pltpu.make_async_copy(srcB.at[pl.ds(rbase, RCH)],
                                       src_v.at[pl.ds(RCH, RCH)], sem_in)
            d3 = pltpu.make_async_copy(dstA.at[pl.ds(rbase, RCH)],
                                       dst_v.at[b, pl.ds(0, RCH)], sem_in)
            d4 = pltpu.make_async_copy(dstB.at[pl.ds(rbase, RCH)],
                                       dst_v.at[b, pl.ds(RCH, RCH)], sem_in)
            d5 = pltpu.make_async_copy(edata.at[L, pl.ds(rbase, RCH)], ed_v, sem_in)
            d1.start(); d2.start(); d3.start(); d4.start(); d5.start()
            d1.wait(); d2.wait()
            gA = pltpu.async_copy(nmp.at[src_v.at[pl.ds(0, RCH)]],
                                  nm_v.at[pl.ds(0, RCH)], sem_g)
            gB = pltpu.async_copy(nmp.at[src_v.at[pl.ds(RCH, RCH)]],
                                  nm_v.at[pl.ds(RCH, RCH)], sem_g)
            d3.wait(); d4.wait(); d5.wait()
            gA.wait()
            gB.wait()

            @pl.when(k > 0)
            def _():
                pltpu.make_async_copy(out_v, Sacc.at[dst_v.at[b]], sem_s).wait()

            for g in range(CH // 16):
                lanes = lanes0 + g * 16
                dstg = plsc.load_gather(dst_v, [bv, lanes])
                isB = (lanes >= RCH).astype(jnp.int32)
                edrow = lanes - isB * RCH
                colb = isB * 64
                ai0 = plsc.load_gather(tab_v, [dstg])
                ai1 = plsc.load_gather(tab_v, [dstg + N])
                aj0 = plsc.load_gather(nm_v, [lanes, jnp.full((16,), C, jnp.int32)])
                aj1 = plsc.load_gather(nm_v, [lanes, jnp.full((16,), C + 1, jnp.int32)])
                e0 = plsc.load_gather(ed_v, [edrow, colb + 32])
                e1 = plsc.load_gather(ed_v, [edrow, colb + 33])
                ex0 = jnp.exp(_lr(ai0 + aj0 + e0) - c0)
                ex1 = jnp.exp(_lr(ai1 + aj1 + e1) - c1)
                for c2 in range(C):
                    fc = jnp.full((16,), c2, jnp.int32)
                    m = (plsc.load_gather(ed_v, [edrow, colb + c2])
                         + plsc.load_gather(nm_v, [lanes, fc]))
                    plsc.store_scatter(out_v, [lanes, fc], m * ex0)
                    plsc.store_scatter(out_v, [lanes, jnp.full((16,), C + c2, jnp.int32)],
                                       m * ex1)
                plsc.store_scatter(out_v, [lanes, jnp.full((16,), 2 * C, jnp.int32)], ex0)
                plsc.store_scatter(out_v, [lanes, jnp.full((16,), 2 * C + 1, jnp.int32)], ex1)
            pltpu.async_copy(out_v, Sacc.at[dst_v.at[b]], sem_s, add=True)
            return ()

        lax.fori_loop(0, NCHUNK, chunk, ())
        pltpu.make_async_copy(out_v, Sacc.at[dst_v.at[0]], sem_s).wait()
        plsc.subcore_barrier()
        pltpu.sync_copy(Sacc.at[pl.ds(sub * NPS, NPS)], out.at[wid])

    return body


def _sc_edge(L, srcA, srcB, dstA, dstB, edata, nmp, tabf, cvec, zrows):
    f = pl.kernel(
        _sc_edge_body(L),
        out_type=jax.ShapeDtypeStruct((NW, NPS, ROWW), jnp.float32),
        mesh=plsc.VectorSubcoreMesh(core_axis_name="c", subcore_axis_name="s"),
        compiler_params=pltpu.CompilerParams(needs_layout_passes=False),
        scratch_types=[
            pltpu.VMEM_SHARED((NPAD, ROWW), jnp.float32),
            pltpu.VMEM((2 * N,), jnp.float32),
            pltpu.VMEM((32,), jnp.float32),
            pltpu.VMEM((CH,), jnp.int32),
            pltpu.VMEM((2, CH), jnp.int32),
            pltpu.VMEM((RCH, 128), jnp.float32),
            pltpu.VMEM((CH, 128), jnp.float32),
            pltpu.VMEM((CH, ROWW), jnp.float32),
            pltpu.SemaphoreType.DMA,
            pltpu.SemaphoreType.DMA,
            pltpu.SemaphoreType.DMA,
        ],
    )
    return f(srcA, srcB, dstA, dstB, edata, nmp, tabf, cvec, zrows)


def kernel(x, edge_index, edge_types, edge_attr, params):
    src, dst = edge_index[0], edge_index[1]
    srcA, srcB = src[:E2], src[E2:]
    dstA, dstB = dst[:E2], dst[E2:]
    proj_w, proj_b = params["proj_w"], params["proj_b"]

    folded = []
    for l in range(3):
        p = params["convs"][l]
        att_w = p["att_w"]
        folded.append(dict(
            hl_w0=p["hl_w"][0], hl_b0=p["hl_b"][0],
            att_wi=att_w[:32], att_wj=att_w[32:64],
            t_att=_lr(p["ete"]) @ att_w[64:80],
            att_wa=att_w[80:96],
            lin_n=p["lin_w"][:32], lin_e=p["lin_w"][32:48],
            W=proj_w @ p["eae_w"],
            c=proj_b @ p["eae_w"],
        ))

    W_all = jnp.concatenate([f["W"] for f in folded], axis=1)
    c_all = jnp.concatenate([f["c"] for f in folded])[None, :]
    M_all = jnp.stack([jnp.concatenate([f["att_wa"], f["lin_e"]], axis=1)
                       for f in folded])
    T_all = jnp.concatenate([f["t_att"] for f in folded], axis=1)

    edata, mx = _p1(edge_attr, edge_types, W_all, c_all, M_all, T_all)
    maxE = mx[0]

    xw0_w = proj_w @ folded[0]["hl_w0"]
    xw0_b = proj_b @ folded[0]["hl_w0"] + folded[0]["hl_b0"]
    zrows = jnp.zeros((NPS, ROWW), jnp.float32)

    h = None
    for l in range(3):
        f = folded[l]
        if l == 0:
            xw = x @ xw0_w + xw0_b
        else:
            xw = h @ f["hl_w0"] + f["hl_b0"]
        ai = xw @ f["att_wi"]
        aj = xw @ f["att_wj"]
        n_m = xw @ f["lin_n"]
        bound = _lr(jnp.max(ai, 0) + jnp.max(aj, 0) + maxE[2 * l:2 * l + 2])

        tabf = jnp.concatenate([ai[:, 0], ai[:, 1]])
        cvec = jnp.repeat(bound, 16)
        nmp = jnp.concatenate(
            [n_m, aj, jnp.zeros((N, 128 - C - 2), jnp.float32)], axis=1)

        Spart = _sc_edge(l, srcA, srcB, dstA, dstB, edata, nmp, tabf,
                         cvec, zrows)
        Spart = Spart.reshape(NC, NPAD, ROWW)
        Ssum = (Spart[0] + Spart[1])[:N]
        D = Ssum[:, 2 * C:2 * C + 2]
        agg = (Ssum[:, :2 * C].reshape(N, 2, 32)
               / (D[:, :, None] + 1e-16) + xw[:, None, :])
        hn = agg.reshape(N, 64)
        if l < 2:
            bn = params["bns"][l]
            mu = hn.mean(0)
            var = hn.var(0)
            hn = bn["g"] * (hn - mu) / jnp.sqrt(var + 1e-5) + bn["b"]
            hn = jax.nn.relu(hn)
        h = hn
    return h.reshape(N, 2, 32)


# speedup vs baseline: 4.7476x; 1.0549x over previous
import functools

import jax
import jax.numpy as jnp
from jax import lax
from jax.experimental import pallas as pl
from jax.experimental.pallas import tpu as pltpu
from jax.experimental.pallas import tpu_sc as plsc

N = 10000
E = 320000
E2 = E // 2
IN_CH = 128
PROJ = 64
C = 32
HEADS = 2
NUM_EDGE_TYPES = 8

BE2 = 2000
GE = E2 // BE2


def _lr(x, s=0.2):
    return jnp.where(x >= 0, x, s * x)


def _p1_half(A, types, W_ref, c_ref, M_ref, T_ref):
    u = _lr(jnp.dot(A, W_ref[...], preferred_element_type=jnp.float32)
            + c_ref[...])
    oh = (types[:, None] ==
          lax.broadcasted_iota(jnp.int32, (BE2, NUM_EDGE_TYPES), 1)
          ).astype(jnp.float32)
    t_att = jnp.dot(oh, T_ref[...], preferred_element_type=jnp.float32)
    ems, eatts = [], []
    for l in range(3):
        ul = u[:, 16 * l:16 * l + 16]
        o = jnp.dot(ul, M_ref[l], preferred_element_type=jnp.float32)
        ems.append(o[:, 2:34])
        eatts.append(o[:, 0:2] + t_att[:, 2 * l:2 * l + 2])
    return ems, eatts


def _p1_body(eaA_ref, eaB_ref, etA_ref, etB_ref, W_ref, c_ref, M_ref, T_ref,
             edata_ref, mx_ref):
    step = pl.program_id(0)
    emsA, eattsA = _p1_half(eaA_ref[...], etA_ref[0, 0], W_ref, c_ref, M_ref, T_ref)
    emsB, eattsB = _p1_half(eaB_ref[...], etB_ref[0, 0], W_ref, c_ref, M_ref, T_ref)
    bm = []
    for l in range(3):
        edata_ref[l, :, 0:32] = emsA[l]
        edata_ref[l, :, 32:34] = eattsA[l]
        edata_ref[l, :, 64:96] = emsB[l]
        edata_ref[l, :, 96:98] = eattsB[l]
        bm.append(jnp.maximum(jnp.max(eattsA[l], axis=0), jnp.max(eattsB[l], axis=0)))
    bmax = jnp.concatenate(bm + [jnp.zeros((2,), jnp.float32)])[None, :]

    @pl.when(step == 0)
    def _():
        mx_ref[...] = bmax

    @pl.when(step != 0)
    def _():
        mx_ref[...] = jnp.maximum(mx_ref[...], bmax)


def _p1(edge_attr, edge_types, W_all, c_all, M_all, T_all):
    etA = edge_types[:E2].reshape(GE, 1, BE2)
    etB = edge_types[E2:].reshape(GE, 1, BE2)
    return pl.pallas_call(
        _p1_body,
        grid=(GE,),
        in_specs=[
            pl.BlockSpec((BE2, IN_CH), lambda i: (i, 0)),
            pl.BlockSpec((BE2, IN_CH), lambda i: (i + GE, 0)),
            pl.BlockSpec((1, 1, BE2), lambda i: (i, 0, 0)),
            pl.BlockSpec((1, 1, BE2), lambda i: (i, 0, 0)),
            pl.BlockSpec((IN_CH, 48), lambda i: (0, 0)),
            pl.BlockSpec((1, 48), lambda i: (0, 0)),
            pl.BlockSpec((3, 16, 34), lambda i: (0, 0, 0)),
            pl.BlockSpec((NUM_EDGE_TYPES, 6), lambda i: (0, 0)),
        ],
        out_specs=[
            pl.BlockSpec((3, BE2, 128), lambda i: (0, i, 0)),
            pl.BlockSpec((1, 8), lambda i: (0, 0)),
        ],
        out_shape=[
            jax.ShapeDtypeStruct((3, E2, 128), jnp.float32),
            jax.ShapeDtypeStruct((1, 8), jnp.float32),
        ],
    )(edge_attr, edge_attr, etA, etB, W_all, c_all, M_all, T_all)


NC = 2
NS = 16
NW = NC * NS
RPW = E2 // NW
RCH = 40
NCHUNK = RPW // RCH
CH = 2 * RCH
ROWW = 128
NPS = 632
NPAD = NS * NPS


def _sc_edge_body(L):
    def body(srcA, srcB, dstA, dstB, edata, nmp, tabf, cvec, zrows, out,
             Sacc, tab_v, cvec_v, src_v, dst_v, ed_v, out_v,
             sem_in, sem_g, sem_s):
        core = lax.axis_index("c")
        sub = lax.axis_index("s")
        wid = core * NS + sub
        rbase0 = wid * RPW

        pltpu.sync_copy(zrows, Sacc.at[pl.ds(sub * NPS, NPS)])
        pltpu.sync_copy(tabf, tab_v)
        pltpu.sync_copy(cvec, cvec_v)
        plsc.subcore_barrier()

        def idx_copies(k, slot):
            rbase = rbase0 + k * RCH
            return (
                pltpu.make_async_copy(srcA.at[pl.ds(rbase, RCH)],
                                      src_v.at[slot, pl.ds(0, RCH)], sem_in),
                pltpu.make_async_copy(srcB.at[pl.ds(rbase, RCH)],
                                      src_v.at[slot, pl.ds(RCH, RCH)], sem_in),
                pltpu.make_async_copy(dstA.at[pl.ds(rbase, RCH)],
                                      dst_v.at[slot, pl.ds(0, RCH)], sem_in),
                pltpu.make_async_copy(dstB.at[pl.ds(rbase, RCH)],
                                      dst_v.at[slot, pl.ds(RCH, RCH)], sem_in),
                pltpu.make_async_copy(edata.at[L, pl.ds(rbase, RCH)],
                                      ed_v.at[slot], sem_in),
            )

        def gathers(slot):
            return (
                pltpu.async_copy(nmp.at[src_v.at[slot, pl.ds(0, RCH)]],
                                 out_v.at[slot, pl.ds(0, RCH)], sem_g),
                pltpu.async_copy(nmp.at[src_v.at[slot, pl.ds(RCH, RCH)]],
                                 out_v.at[slot, pl.ds(RCH, RCH)], sem_g),
            )

        for d in idx_copies(0, 0):
            d.start()
            d.wait()
        for gd in gathers(0):
            gd.wait()

        def chunk(k, _):
            lanes0 = lax.iota(jnp.int32, 16)
            c0 = cvec_v[pl.ds(0, 16)]
            c1 = cvec_v[pl.ds(16, 16)]
            p = lax.rem(k, 2)
            q = 1 - p
            pv = jnp.zeros((16,), jnp.int32) + p

            @pl.when(k > 0)
            def _():
                pltpu.make_async_copy(out_v.at[q], Sacc.at[dst_v.at[q]], sem_s).wait()

            @pl.when(k + 1 < NCHUNK)
            def _():
                for d in idx_copies(k + 1, q):
                    d.start()

            @pl.when(k > 0)
            def _():
                pltpu.make_async_copy(nmp.at[src_v.at[p, pl.ds(0, RCH)]],
                                      out_v.at[p, pl.ds(0, RCH)], sem_g).wait()
                pltpu.make_async_copy(nmp.at[src_v.at[p, pl.ds(RCH, RCH)]],
                                      out_v.at[p, pl.ds(RCH, RCH)], sem_g).wait()

            for g in range(CH // 16):
                lanes = lanes0 + g * 16
                dstg = plsc.load_gather(dst_v, [pv, lanes])
                isB = (lanes >= RCH).astype(jnp.int32)
                edrow = lanes - isB * RCH
                colb = isB * 64
                av = plsc.bitcast(plsc.load_gather(tab_v, [dstg]), jnp.int32)
                ai0 = plsc.bitcast((av >> 16) << 16, jnp.float32)
                ai1 = plsc.bitcast(av << 16, jnp.float32)
                aj0 = plsc.load_gather(out_v, [pv, lanes, jnp.full((16,), C, jnp.int32)])
                aj1 = plsc.load_gather(out_v, [pv, lanes, jnp.full((16,), C + 1, jnp.int32)])
                e0 = plsc.load_gather(ed_v, [pv, edrow, colb + 32])
                e1 = plsc.load_gather(ed_v, [pv, edrow, colb + 33])
                ex0 = jnp.exp(_lr(ai0 + aj0 + e0) - c0)
                ex1 = jnp.exp(_lr(ai1 + aj1 + e1) - c1)
                for c2 in range(C):
                    fc = jnp.full((16,), c2, jnp.int32)
                    m = (plsc.load_gather(ed_v, [pv, edrow, colb + c2])
                         + plsc.load_gather(out_v, [pv, lanes, fc]))
                    plsc.store_scatter(out_v, [pv, lanes, fc], m * ex0)
                    plsc.store_scatter(out_v,
                                      [pv, lanes, jnp.full((16,), C + c2, jnp.int32)],
                                      m * ex1)
                plsc.store_scatter(out_v, [pv, lanes, jnp.full((16,), 2 * C, jnp.int32)],
                                   ex0)
                plsc.store_scatter(out_v,
                                   [pv, lanes, jnp.full((16,), 2 * C + 1, jnp.int32)],
                                   ex1)

            @pl.when(k + 1 < NCHUNK)
            def _():
                for d in idx_copies(k + 1, q):
                    d.wait()
                gathers(q)

            pltpu.async_copy(out_v.at[p], Sacc.at[dst_v.at[p]], sem_s, add=True)
            return ()

        lax.fori_loop(0, NCHUNK, chunk, ())
        pltpu.make_async_copy(out_v.at[0], Sacc.at[dst_v.at[0]], sem_s).wait()
        plsc.subcore_barrier()
        pltpu.sync_copy(Sacc.at[pl.ds(sub * NPS, NPS)], out.at[wid])

    return body


def _sc_edge(L, srcA, srcB, dstA, dstB, edata, nmp, tabf, cvec, zrows):
    f = pl.kernel(
        _sc_edge_body(L),
        out_type=jax.ShapeDtypeStruct((NW, NPS, ROWW), jnp.float32),
        mesh=plsc.VectorSubcoreMesh(core_axis_name="c", subcore_axis_name="s"),
        compiler_params=pltpu.CompilerParams(needs_layout_passes=False),
        scratch_types=[
            pltpu.VMEM_SHARED((NPAD, ROWW), jnp.float32),
            pltpu.VMEM((N,), jnp.float32),
            pltpu.VMEM((32,), jnp.float32),
            pltpu.VMEM((2, CH), jnp.int32),
            pltpu.VMEM((2, CH), jnp.int32),
            pltpu.VMEM((2, RCH, 128), jnp.float32),
            pltpu.VMEM((2, CH, ROWW), jnp.float32),
            pltpu.SemaphoreType.DMA,
            pltpu.SemaphoreType.DMA,
            pltpu.SemaphoreType.DMA,
        ],
    )
    return f(srcA, srcB, dstA, dstB, edata, nmp, tabf, cvec, zrows)


def kernel(x, edge_index, edge_types, edge_attr, params):
    src, dst = edge_index[0], edge_index[1]
    srcA, srcB = src[:E2], src[E2:]
    dstA, dstB = dst[:E2], dst[E2:]
    proj_w, proj_b = params["proj_w"], params["proj_b"]

    folded = []
    for l in range(3):
        p = params["convs"][l]
        att_w = p["att_w"]
        folded.append(dict(
            hl_w0=p["hl_w"][0], hl_b0=p["hl_b"][0],
            att_wi=att_w[:32], att_wj=att_w[32:64],
            t_att=_lr(p["ete"]) @ att_w[64:80],
            att_wa=att_w[80:96],
            lin_n=p["lin_w"][:32], lin_e=p["lin_w"][32:48],
            W=proj_w @ p["eae_w"],
            c=proj_b @ p["eae_w"],
        ))

    W_all = jnp.concatenate([f["W"] for f in folded], axis=1)
    c_all = jnp.concatenate([f["c"] for f in folded])[None, :]
    M_all = jnp.stack([jnp.concatenate([f["att_wa"], f["lin_e"]], axis=1)
                       for f in folded])
    T_all = jnp.concatenate([f["t_att"] for f in folded], axis=1)

    edata, mx = _p1(edge_attr, edge_types, W_all, c_all, M_all, T_all)
    maxE = mx[0]

    xw0_w = proj_w @ folded[0]["hl_w0"]
    xw0_b = proj_b @ folded[0]["hl_w0"] + folded[0]["hl_b0"]
    zrows = jnp.zeros((NPS, ROWW), jnp.float32)

    h = None
    for l in range(3):
        f = folded[l]
        if l == 0:
            xw = x @ xw0_w + xw0_b
        else:
            xw = h @ f["hl_w0"] + f["hl_b0"]
        ai = xw @ f["att_wi"]
        aj = xw @ f["att_wj"]
        n_m = xw @ f["lin_n"]
        bound = _lr(jnp.max(ai, 0) + jnp.max(aj, 0) + maxE[2 * l:2 * l + 2])

        a0 = jax.lax.bitcast_convert_type(
            ai[:, 0].astype(jnp.bfloat16).astype(jnp.float32), jnp.uint32)
        a1 = jax.lax.bitcast_convert_type(
            ai[:, 1].astype(jnp.bfloat16).astype(jnp.float32), jnp.uint32)
        tabf = jax.lax.bitcast_convert_type(a0 | (a1 >> 16), jnp.float32)
        cvec = jnp.repeat(bound, 16)
        nmp = jnp.concatenate(
            [n_m, aj, jnp.zeros((N, 128 - C - 2), jnp.float32)], axis=1)

        Spart = _sc_edge(l, srcA, srcB, dstA, dstB, edata, nmp, tabf,
                         cvec, zrows)
        Spart = Spart.reshape(NC, NPAD, ROWW)
        Ssum = (Spart[0] + Spart[1])[:N]
        D = Ssum[:, 2 * C:2 * C + 2]
        agg = (Ssum[:, :2 * C].reshape(N, 2, 32)
               / (D[:, :, None] + 1e-16) + xw[:, None, :])
        hn = agg.reshape(N, 64)
        if l < 2:
            bn = params["bns"][l]
            mu = hn.mean(0)
            var = hn.var(0)
            hn = bn["g"] * (hn - mu) / jnp.sqrt(var + 1e-5) + bn["b"]
            hn = jax.nn.relu(hn)
        h = hn
    return h.reshape(N, 2, 32)
